# baseline (speedup 1.0000x reference)
import functools

import jax
import jax.numpy as jnp
from jax import lax
from jax.experimental import pallas as pl
from jax.experimental.pallas import tpu as pltpu
from jax.experimental.pallas import tpu_sc as plsc

N_NODES = 10000
E = 320000
D_IN = 128
D_MP = 64

NT = 16
TE = E // NT
C = 80
NCHUNK = TE // C
SB = 50
NBLK = NCHUNK // SB
ZR = 80
NZCHUNK = N_NODES // ZR


def _proj_body(xa_ref, xp_ref, wa_ref, wp_ref, ha_ref, hp_ref):
    dn = (((1,), (1,)), ((), ()))
    ha_ref[...] = lax.dot_general(
        xa_ref[...], wa_ref[...], dn,
        preferred_element_type=jnp.float32)
    hp_ref[...] = lax.dot_general(
        xp_ref[...], wp_ref[...], dn,
        preferred_element_type=jnp.float32)


def _project(x_author, x_paper, W_author, W_paper):
    blk = 1000
    grid = (N_NODES // blk,)
    return pl.pallas_call(
        _proj_body,
        grid=grid,
        in_specs=[
            pl.BlockSpec((blk, D_IN), lambda i: (i, 0)),
            pl.BlockSpec((blk, D_IN), lambda i: (i, 0)),
            pl.BlockSpec((D_MP, D_IN), lambda i: (0, 0)),
            pl.BlockSpec((D_MP, D_IN), lambda i: (0, 0)),
        ],
        out_specs=[
            pl.BlockSpec((blk, D_MP), lambda i: (i, 0)),
            pl.BlockSpec((blk, D_MP), lambda i: (i, 0)),
        ],
        out_shape=[
            jax.ShapeDtypeStruct((N_NODES, D_MP), jnp.float32),
            jax.ShapeDtypeStruct((N_NODES, D_MP), jnp.float32),
        ],
    )(x_author, x_paper, W_author, W_paper)


def _bcast_lane(v, k):
    idx = jnp.full((16, 1), k, dtype=jnp.int32)
    dnums = lax.GatherDimensionNumbers(
        offset_dims=(), collapsed_slice_dims=(0,), start_index_map=(0,))
    return lax.gather(v, idx, dnums, (1,),
                      mode=lax.GatherScatterMode.PROMISE_IN_BOUNDS)


def _scale(gbuf, sbuf, st_w, c):
    for g in range(C // 16):
        w16 = st_w[c, pl.ds(16 * g, 16)]
        for k in range(16):
            e = 16 * g + k
            w = _bcast_lane(w16, k)
            for d in range(D_MP // 16):
                sl = pl.ds(16 * d, 16)
                sbuf[e, sl] = gbuf[e, sl] * w


def _zero_gb0(gb):
    def zero_body(i, carry):
        zeros = jnp.zeros((16,), jnp.float32)
        for d in range(D_MP // 16):
            gb[0][i, pl.ds(16 * d, 16)] = zeros
        return carry

    lax.fori_loop(0, ZR, zero_body, 0)


def _hop(row_hbm, col_hbm, w_hbm, table, acc, st, gb, sb, semg, sems, semt,
         s):
    st_row, st_col, st_w = st

    def block_body(b, carry):
        row_src = row_hbm.at[s, pl.ds(b * SB, SB)]
        col_src = col_hbm.at[s, pl.ds(b * SB, SB)]
        w_src = w_hbm.at[s, pl.ds(b * SB, SB)]
        pltpu.async_copy(row_src, st_row, semg[2])
        pltpu.async_copy(col_src, st_col, semt)
        pltpu.async_copy(w_src, st_w, semg[3])
        pltpu.make_async_copy(row_src, st_row, semg[2]).wait()
        pltpu.make_async_copy(w_src, st_w, semg[3]).wait()
        pltpu.make_async_copy(col_src, st_col, semt).wait()
        pltpu.async_copy(table.at[st_col.at[0]], gb[0], semg[0])
        pltpu.async_copy(table.at[st_col.at[1]], gb[1], semg[1])
        pltpu.async_copy(table.at[st_col.at[2]], gb[2], semg[2])
        pltpu.async_copy(table.at[st_col.at[3]], gb[3], semg[3])

        def halfpair(c0, x, y, pp):
            pltpu.make_async_copy(table.at[st_col.at[c0]], gb[x],
                                  semg[x]).wait()

            @pl.when(pp > 0)
            def _():
                pltpu.make_async_copy(sb[y], acc.at[st_row.at[c0]],
                                      sems[y]).wait()

            _scale(gb[x], sb[y], st_w, c0)
            pltpu.async_copy(sb[y], acc.at[st_row.at[c0]], sems[y], add=True)

            @pl.when(c0 + 4 < SB)
            def _():
                pltpu.async_copy(table.at[st_col.at[c0 + 4]], gb[x], semg[x])

        def pair_body(pp, carry2):
            c0 = 2 * pp

            @pl.when(pp % 2 == 0)
            def _():
                halfpair(c0, 0, 0, pp)
                halfpair(c0 + 1, 1, 1, pp)

            @pl.when(pp % 2 == 1)
            def _():
                halfpair(c0, 2, 0, pp)
                halfpair(c0 + 1, 3, 1, pp)

            return carry2

        lax.fori_loop(0, SB // 2, pair_body, 0)
        pltpu.make_async_copy(sb[0], acc.at[st_row.at[0]], sems[0]).wait()
        pltpu.make_async_copy(sb[1], acc.at[st_row.at[1]], sems[1]).wait()
        return carry

    lax.fori_loop(0, NBLK, block_body, 0)


def _rr_copies(s, issue):
    for wait in (False, True):
        for j in range((NZCHUNK + NT - 1) // NT):
            idx = s + NT * j

            @pl.when(idx < NZCHUNK)
            def _():
                issue(idx * ZR, wait)


def _chain(h_hbm, e1, e2, out_hbm, accA, accB, st, gb, sb, semg, sems, semt,
           s):
    _zero_gb0(gb)

    def stage_and_zero(off, wait):
        cp1 = pltpu.make_async_copy(h_hbm.at[pl.ds(off, ZR)],
                                    accA.at[pl.ds(off, ZR)], semt)
        cp2 = pltpu.make_async_copy(gb[0], accB.at[pl.ds(off, ZR)], semg[0])
        if wait:
            cp1.wait()
            cp2.wait()
        else:
            cp1.start()
            cp2.start()

    _rr_copies(s, stage_and_zero)
    plsc.subcore_barrier()

    _hop(*e1, accA, accB, st, gb, sb, semg, sems, semt, s)
    plsc.subcore_barrier()

    _zero_gb0(gb)

    def clear(off, wait):
        cp = pltpu.make_async_copy(gb[0], accA.at[pl.ds(off, ZR)], semt)
        if wait:
            cp.wait()
        else:
            cp.start()

    _rr_copies(s, clear)
    plsc.subcore_barrier()

    _hop(*e2, accB, accA, st, gb, sb, semg, sems, semt, s)
    plsc.subcore_barrier()

    def dump(off, wait):
        cp = pltpu.make_async_copy(accA.at[pl.ds(off, ZR)],
                                   out_hbm.at[pl.ds(off, ZR)], semt)
        if wait:
            cp.wait()
        else:
            cp.start()

    _rr_copies(s, dump)


def _sc_body(ha, hp, row_w, col_w, w_w, row_r, col_r, w_r, apa, pap,
             accA, accB, st_row, st_col, st_w,
             gb0, gb1, gb2, gb3, sb0, sb1,
             semg0, semg1, semg2, semg3, sems0, sems1, semt):
    cid = lax.axis_index("c")
    s = lax.axis_index("s")
    writes = (row_w, col_w, w_w)
    written = (row_r, col_r, w_r)
    st = (st_row, st_col, st_w)
    gb = (gb0, gb1, gb2, gb3)
    sb = (sb0, sb1)
    semg = (semg0, semg1, semg2, semg3)
    sems = (sems0, sems1)

    @pl.when(cid == 0)
    def _():
        _chain(ha, writes, written, apa, accA, accB, st, gb, sb,
               semg, sems, semt, s)

    @pl.when(cid == 1)
    def _():
        _chain(hp, written, writes, pap, accA, accB, st, gb, sb,
               semg, sems, semt, s)


def _metapaths(ha, hp, row_w, col_w, w_w, row_r, col_r, w_r):
    mesh = plsc.VectorSubcoreMesh(core_axis_name="c", subcore_axis_name="s")
    f = pl.kernel(
        _sc_body,
        out_type=[
            jax.ShapeDtypeStruct((N_NODES, D_MP), jnp.float32),
            jax.ShapeDtypeStruct((N_NODES, D_MP), jnp.float32),
        ],
        mesh=mesh,
        scratch_types=[
            pltpu.VMEM_SHARED((N_NODES, D_MP), jnp.float32),
            pltpu.VMEM_SHARED((N_NODES, D_MP), jnp.float32),
            pltpu.VMEM((SB, C), jnp.int32),
            pltpu.VMEM((SB, C), jnp.int32),
            pltpu.VMEM((SB, C), jnp.float32),
            pltpu.VMEM((C, D_MP), jnp.float32),
            pltpu.VMEM((C, D_MP), jnp.float32),
            pltpu.VMEM((C, D_MP), jnp.float32),
            pltpu.VMEM((C, D_MP), jnp.float32),
            pltpu.VMEM((C, D_MP), jnp.float32),
            pltpu.VMEM((C, D_MP), jnp.float32),
            pltpu.SemaphoreType.DMA,
            pltpu.SemaphoreType.DMA,
            pltpu.SemaphoreType.DMA,
            pltpu.SemaphoreType.DMA,
            pltpu.SemaphoreType.DMA,
            pltpu.SemaphoreType.DMA,
            pltpu.SemaphoreType.DMA,
        ],
        compiler_params=pltpu.CompilerParams(use_tc_tiling_on_sc=False),
    )
    return f(ha, hp, row_w, col_w, w_w, row_r, col_r, w_r)


def kernel(x_author, x_paper, edge_index_writes, edge_w_writes,
           edge_index_written, edge_w_written, W_author, W_paper):
    ha, hp = _project(x_author, x_paper, W_author, W_paper)
    row_w = edge_index_writes[0].reshape(NT, NCHUNK, C)
    col_w = edge_index_writes[1].reshape(NT, NCHUNK, C)
    w_w = edge_w_writes.reshape(NT, NCHUNK, C)
    row_r = edge_index_written[0].reshape(NT, NCHUNK, C)
    col_r = edge_index_written[1].reshape(NT, NCHUNK, C)
    w_r = edge_w_written.reshape(NT, NCHUNK, C)
    apa, pap = _metapaths(ha, hp, row_w, col_w, w_w, row_r, col_r, w_r)
    return (apa, pap)


# speedup vs baseline: 1.0167x; 1.0167x over previous
import functools

import jax
import jax.numpy as jnp
from jax import lax
from jax.experimental import pallas as pl
from jax.experimental.pallas import tpu as pltpu
from jax.experimental.pallas import tpu_sc as plsc

N_NODES = 10000
E = 320000
D_IN = 128
D_MP = 64

NT = 16
TE = E // NT
C = 80
NCHUNK = TE // C
SB = 50
NBLK = NCHUNK // SB
ZR = 80
NZCHUNK = N_NODES // ZR


def _proj_body(xa_ref, xp_ref, wa_ref, wp_ref, ha_ref, hp_ref):
    dn = (((1,), (1,)), ((), ()))
    ha_ref[...] = lax.dot_general(
        xa_ref[...], wa_ref[...], dn,
        preferred_element_type=jnp.float32)
    hp_ref[...] = lax.dot_general(
        xp_ref[...], wp_ref[...], dn,
        preferred_element_type=jnp.float32)


def _project(x_author, x_paper, W_author, W_paper):
    blk = 1000
    grid = (N_NODES // blk,)
    return pl.pallas_call(
        _proj_body,
        grid=grid,
        in_specs=[
            pl.BlockSpec((blk, D_IN), lambda i: (i, 0)),
            pl.BlockSpec((blk, D_IN), lambda i: (i, 0)),
            pl.BlockSpec((D_MP, D_IN), lambda i: (0, 0)),
            pl.BlockSpec((D_MP, D_IN), lambda i: (0, 0)),
        ],
        out_specs=[
            pl.BlockSpec((blk, D_MP), lambda i: (i, 0)),
            pl.BlockSpec((blk, D_MP), lambda i: (i, 0)),
        ],
        out_shape=[
            jax.ShapeDtypeStruct((N_NODES, D_MP), jnp.float32),
            jax.ShapeDtypeStruct((N_NODES, D_MP), jnp.float32),
        ],
    )(x_author, x_paper, W_author, W_paper)


def _bcast_lane(v, k):
    idx = jnp.full((16, 1), k, dtype=jnp.int32)
    dnums = lax.GatherDimensionNumbers(
        offset_dims=(), collapsed_slice_dims=(0,), start_index_map=(0,))
    return lax.gather(v, idx, dnums, (1,),
                      mode=lax.GatherScatterMode.PROMISE_IN_BOUNDS)


def _scale(gbuf, sbuf, st_w, c):
    for g in range(C // 16):
        w16 = st_w[c, pl.ds(16 * g, 16)]
        for k in range(16):
            e = 16 * g + k
            w = _bcast_lane(w16, k)
            for d in range(D_MP // 16):
                sl = pl.ds(16 * d, 16)
                sbuf[e, sl] = gbuf[e, sl] * w


def _zero_gb0(gb):
    def zero_body(i, carry):
        zeros = jnp.zeros((16,), jnp.float32)
        for d in range(D_MP // 16):
            gb[0][i, pl.ds(16 * d, 16)] = zeros
        return carry

    lax.fori_loop(0, ZR, zero_body, 0)


def _hop(row_hbm, col_hbm, w_hbm, table, acc, st, gb, sb, semg, sems, semt,
         s):
    st_row, st_col, st_w = st

    def block_body(b, carry):
        row_src = row_hbm.at[s, pl.ds(b * SB, SB)]
        col_src = col_hbm.at[s, pl.ds(b * SB, SB)]
        w_src = w_hbm.at[s, pl.ds(b * SB, SB)]
        pltpu.async_copy(row_src, st_row, semg[2])
        pltpu.async_copy(col_src, st_col, semt)
        pltpu.async_copy(w_src, st_w, semg[3])
        pltpu.make_async_copy(col_src, st_col, semt).wait()
        pltpu.async_copy(table.at[st_col.at[0]], gb[0], semg[0])
        pltpu.async_copy(table.at[st_col.at[1]], gb[1], semg[1])
        pltpu.make_async_copy(row_src, st_row, semg[2]).wait()
        pltpu.make_async_copy(w_src, st_w, semg[3]).wait()

        def halfpair(c0, x, z, y, pp):
            pltpu.make_async_copy(table.at[st_col.at[c0]], gb[x],
                                  semg[x]).wait()

            @pl.when(pp > 0)
            def _():
                pltpu.make_async_copy(sb[y], acc.at[st_row.at[c0]],
                                      sems[y]).wait()

            _scale(gb[x], sb[y], st_w, c0)
            pltpu.async_copy(sb[y], acc.at[st_row.at[c0]], sems[y], add=True)

            @pl.when(c0 + 2 < SB)
            def _():
                pltpu.async_copy(table.at[st_col.at[c0 + 2]], gb[z], semg[z])

        def pair_body(pp, carry2):
            c0 = 2 * pp

            @pl.when(pp % 2 == 0)
            def _():
                halfpair(c0, 0, 2, 0, pp)
                halfpair(c0 + 1, 1, 3, 1, pp)

            @pl.when(pp % 2 == 1)
            def _():
                halfpair(c0, 2, 0, 0, pp)
                halfpair(c0 + 1, 3, 1, 1, pp)

            return carry2

        lax.fori_loop(0, SB // 2, pair_body, 0)
        pltpu.make_async_copy(sb[0], acc.at[st_row.at[0]], sems[0]).wait()
        pltpu.make_async_copy(sb[1], acc.at[st_row.at[1]], sems[1]).wait()
        return carry

    lax.fori_loop(0, NBLK, block_body, 0)


def _rr_copies(s, issue):
    for wait in (False, True):
        for j in range((NZCHUNK + NT - 1) // NT):
            idx = s + NT * j

            @pl.when(idx < NZCHUNK)
            def _():
                issue(idx * ZR, wait)


def _chain(h_hbm, e1, e2, out_hbm, accA, accB, st, gb, sb, semg, sems, semt,
           s):
    _zero_gb0(gb)

    def stage_and_zero(off, wait):
        cp1 = pltpu.make_async_copy(h_hbm.at[pl.ds(off, ZR)],
                                    accA.at[pl.ds(off, ZR)], semt)
        cp2 = pltpu.make_async_copy(gb[0], accB.at[pl.ds(off, ZR)], semg[0])
        if wait:
            cp1.wait()
            cp2.wait()
        else:
            cp1.start()
            cp2.start()

    _rr_copies(s, stage_and_zero)
    plsc.subcore_barrier()

    _hop(*e1, accA, accB, st, gb, sb, semg, sems, semt, s)
    plsc.subcore_barrier()

    _zero_gb0(gb)

    def clear(off, wait):
        cp = pltpu.make_async_copy(gb[0], accA.at[pl.ds(off, ZR)], semt)
        if wait:
            cp.wait()
        else:
            cp.start()

    _rr_copies(s, clear)
    plsc.subcore_barrier()

    _hop(*e2, accB, accA, st, gb, sb, semg, sems, semt, s)
    plsc.subcore_barrier()

    def dump(off, wait):
        cp = pltpu.make_async_copy(accA.at[pl.ds(off, ZR)],
                                   out_hbm.at[pl.ds(off, ZR)], semt)
        if wait:
            cp.wait()
        else:
            cp.start()

    _rr_copies(s, dump)


def _sc_body(ha, hp, row_w, col_w, w_w, row_r, col_r, w_r, apa, pap,
             accA, accB, st_row, st_col, st_w,
             gb0, gb1, gb2, gb3, sb0, sb1,
             semg0, semg1, semg2, semg3, sems0, sems1, semt):
    cid = lax.axis_index("c")
    s = lax.axis_index("s")
    writes = (row_w, col_w, w_w)
    written = (row_r, col_r, w_r)
    st = (st_row, st_col, st_w)
    gb = (gb0, gb1, gb2, gb3)
    sb = (sb0, sb1)
    semg = (semg0, semg1, semg2, semg3)
    sems = (sems0, sems1)

    @pl.when(cid == 0)
    def _():
        _chain(ha, writes, written, apa, accA, accB, st, gb, sb,
               semg, sems, semt, s)

    @pl.when(cid == 1)
    def _():
        _chain(hp, written, writes, pap, accA, accB, st, gb, sb,
               semg, sems, semt, s)


def _metapaths(ha, hp, row_w, col_w, w_w, row_r, col_r, w_r):
    mesh = plsc.VectorSubcoreMesh(core_axis_name="c", subcore_axis_name="s")
    f = pl.kernel(
        _sc_body,
        out_type=[
            jax.ShapeDtypeStruct((N_NODES, D_MP), jnp.float32),
            jax.ShapeDtypeStruct((N_NODES, D_MP), jnp.float32),
        ],
        mesh=mesh,
        scratch_types=[
            pltpu.VMEM_SHARED((N_NODES, D_MP), jnp.float32),
            pltpu.VMEM_SHARED((N_NODES, D_MP), jnp.float32),
            pltpu.VMEM((SB, C), jnp.int32),
            pltpu.VMEM((SB, C), jnp.int32),
            pltpu.VMEM((SB, C), jnp.float32),
            pltpu.VMEM((C, D_MP), jnp.float32),
            pltpu.VMEM((C, D_MP), jnp.float32),
            pltpu.VMEM((C, D_MP), jnp.float32),
            pltpu.VMEM((C, D_MP), jnp.float32),
            pltpu.VMEM((C, D_MP), jnp.float32),
            pltpu.VMEM((C, D_MP), jnp.float32),
            pltpu.SemaphoreType.DMA,
            pltpu.SemaphoreType.DMA,
            pltpu.SemaphoreType.DMA,
            pltpu.SemaphoreType.DMA,
            pltpu.SemaphoreType.DMA,
            pltpu.SemaphoreType.DMA,
            pltpu.SemaphoreType.DMA,
        ],
        compiler_params=pltpu.CompilerParams(use_tc_tiling_on_sc=False),
    )
    return f(ha, hp, row_w, col_w, w_w, row_r, col_r, w_r)


def kernel(x_author, x_paper, edge_index_writes, edge_w_writes,
           edge_index_written, edge_w_written, W_author, W_paper):
    ha, hp = _project(x_author, x_paper, W_author, W_paper)
    row_w = edge_index_writes[0].reshape(NT, NCHUNK, C)
    col_w = edge_index_writes[1].reshape(NT, NCHUNK, C)
    w_w = edge_w_writes.reshape(NT, NCHUNK, C)
    row_r = edge_index_written[0].reshape(NT, NCHUNK, C)
    col_r = edge_index_written[1].reshape(NT, NCHUNK, C)
    w_r = edge_w_written.reshape(NT, NCHUNK, C)
    apa, pap = _metapaths(ha, hp, row_w, col_w, w_w, row_r, col_r, w_r)
    return (apa, pap)
